# baseline (speedup 1.0000x reference)
import functools

import jax
import jax.numpy as jnp
from jax import lax
from jax.experimental import pallas as pl
from jax.experimental.pallas import tpu as pltpu
from jax.experimental.pallas import tpu_sc as plsc

_LANES = 16


def _dyn_gather(src, idx):
    return lax.gather(
        src,
        idx[:, None],
        lax.GatherDimensionNumbers(
            offset_dims=(), collapsed_slice_dims=(0,), start_index_map=(0,)
        ),
        slice_sizes=(1,),
        mode=lax.GatherScatterMode.PROMISE_IN_BOUNDS,
    )


def _make_topk16(B, N, chunk):
    info = plsc.get_sparse_core_info()
    nw = info.num_cores * info.num_subcores
    rows_per_w = B // nw
    nchunk = N // chunk
    nv = chunk // _LANES

    mesh = plsc.VectorSubcoreMesh(core_axis_name="c", subcore_axis_name="s")

    @functools.partial(
        pl.kernel,
        mesh=mesh,
        out_type=[
            jax.ShapeDtypeStruct((B, _LANES), jnp.float32),
            jax.ShapeDtypeStruct((B, _LANES), jnp.int32),
        ],
        compiler_params=pltpu.CompilerParams(needs_layout_passes=False),
        scratch_types=[
            pltpu.VMEM((chunk,), jnp.float32),
            pltpu.VMEM((rows_per_w, _LANES), jnp.float32),
            pltpu.VMEM((rows_per_w, _LANES), jnp.int32),
            pltpu.SemaphoreType.DMA,
        ],
    )
    def topk16(probs_hbm, outv_hbm, outi_hbm, buf, rv, ri, sem):
        wid = lax.axis_index("s") * info.num_cores + lax.axis_index("c")
        iota = lax.iota(jnp.int32, _LANES)
        shift_idx = jnp.maximum(iota - 1, 0)
        lane15 = jnp.full((_LANES,), _LANES - 1, jnp.int32)

        def insert_loop(v, base, CV, CI, Ts):
            m = v > jnp.full((_LANES,), 0.0, jnp.float32) + Ts

            def wcond(st):
                _, _, _, m = st
                return jnp.any(m)

            def wbody(st):
                CV, CI, Ts, m = st
                fv = plsc.all_reduce_ffs(m)
                x = _dyn_gather(v, fv)
                xi = fv + base
                ge = CV >= x
                p = plsc.all_reduce_population_count(ge)
                sv = _dyn_gather(CV, shift_idx)
                si = _dyn_gather(CI, shift_idx)
                CVn = jnp.where(iota < p, CV, jnp.where(iota == p, x, sv))
                CIn = jnp.where(iota < p, CI, jnp.where(iota == p, xi, si))
                Tn = jnp.min(CVn)
                mn = m & (iota != fv) & (v > jnp.zeros((_LANES,), jnp.float32) + Tn)
                return CVn, CIn, Tn, mn

            CV, CI, Ts, _ = lax.while_loop(wcond, wbody, (CV, CI, Ts, m))
            return CV, CI, Ts

        for r in range(rows_per_w):
            row = wid * rows_per_w + r

            def chunk_body(c, carry, row=row):
                CV, CI, Ts = carry
                off = pl.multiple_of(c * chunk, chunk)
                pltpu.sync_copy(probs_hbm.at[row, pl.ds(off, chunk)], buf)

                def vbody(j, carry2, c=c):
                    CV, CI, Ts = carry2
                    v = buf[pl.ds(pl.multiple_of(j * _LANES, _LANES), _LANES)]
                    mx = jnp.max(v)
                    base = c * chunk + j * _LANES
                    CV, CI, Ts = lax.cond(
                        mx > Ts,
                        lambda a: insert_loop(v, base, *a),
                        lambda a: a,
                        (CV, CI, Ts),
                    )
                    return CV, CI, Ts

                return lax.fori_loop(0, nv, vbody, (CV, CI, Ts))

            CV0 = jnp.full((_LANES,), -1.0, jnp.float32)
            CI0 = jnp.zeros((_LANES,), jnp.int32)
            CV, CI, _ = lax.fori_loop(
                0, nchunk, chunk_body, (CV0, CI0, jnp.float32(-1.0))
            )
            rv[r] = CV
            ri[r] = CI

        out_off = pl.multiple_of(wid * rows_per_w, rows_per_w)
        pltpu.sync_copy(rv, outv_hbm.at[pl.ds(out_off, rows_per_w)])
        pltpu.sync_copy(ri, outi_hbm.at[pl.ds(out_off, rows_per_w)])

    return topk16


def _span_f1(ps, pe, gs, ge):
    ps_f = ps.astype(jnp.float32)
    pe_f = pe.astype(jnp.float32)
    gs_f = gs.astype(jnp.float32)
    ge_f = ge.astype(jnp.float32)
    inter = jnp.maximum(0.0, jnp.minimum(pe_f, ge_f) - jnp.maximum(ps_f, gs_f) + 1.0)
    pred_len = jnp.maximum(pe_f - ps_f + 1.0, 1.0)
    gold_len = jnp.maximum(ge_f - gs_f + 1.0, 1.0)
    prec = inter / pred_len
    rec = inter / gold_len
    f1 = 2.0 * prec * rec / jnp.maximum(prec + rec, 1e-8)
    return jnp.where(ps <= pe, f1, 0.0).astype(jnp.float32)


def kernel(probs, start, end, context_len):
    B, N = probs.shape
    topk16 = _make_topk16(B, N, chunk=16384)
    cv, ci = topk16(probs)

    K = 10
    kbest_probs = cv[:, :K]
    kbest = ci[:, :K]
    greedy_idx = ci[:, 0]
    greedy_reward = _span_f1(
        greedy_idx // context_len, greedy_idx % context_len, start, end
    )
    skey = jax.random.key(42)
    indice = jax.random.categorical(skey, jnp.log(kbest_probs + 1e-20), axis=1)
    sample_idx = jnp.take_along_axis(kbest, indice[:, None], axis=1).squeeze(1)
    sample_reward = _span_f1(
        sample_idx // context_len, sample_idx % context_len, start, end
    )
    greedy_better = jnp.clip(greedy_reward - sample_reward, 0.0, 1.0e7)
    sample_better = jnp.clip(sample_reward, 0.0, 1.0e7)
    greedy_loss = -cv[:, 0]
    sample_loss = -jnp.take_along_axis(kbest_probs, indice[:, None], axis=1).squeeze(1)
    total_loss = greedy_better * greedy_loss + sample_better * sample_loss
    return jnp.mean(total_loss)


# speedup vs baseline: 9.4780x; 9.4780x over previous
import functools

import jax
import jax.numpy as jnp
from jax import lax
from jax.experimental import pallas as pl
from jax.experimental.pallas import tpu as pltpu
from jax.experimental.pallas import tpu_sc as plsc

_LANES = 16


def _dyn_gather(src, idx):
    return lax.gather(
        src,
        idx[:, None],
        lax.GatherDimensionNumbers(
            offset_dims=(), collapsed_slice_dims=(0,), start_index_map=(0,)
        ),
        slice_sizes=(1,),
        mode=lax.GatherScatterMode.PROMISE_IN_BOUNDS,
    )


def _make_topk16(B, N, chunk=32768, sub=8):
    info = plsc.get_sparse_core_info()
    nw = info.num_cores * info.num_subcores
    rows_per_w = B // nw
    chunks_per_row = N // chunk
    totch = rows_per_w * chunks_per_row
    blk = sub * sub * _LANES
    blocks_per_chunk = chunk // blk

    mesh = plsc.VectorSubcoreMesh(core_axis_name="c", subcore_axis_name="s")

    @functools.partial(
        pl.kernel,
        mesh=mesh,
        out_type=[
            jax.ShapeDtypeStruct((B, _LANES), jnp.float32),
            jax.ShapeDtypeStruct((B, _LANES), jnp.int32),
        ],
        compiler_params=pltpu.CompilerParams(needs_layout_passes=False),
        scratch_types=[
            pltpu.VMEM((2, chunk), jnp.float32),
            pltpu.VMEM((sub, _LANES), jnp.float32),
            pltpu.VMEM((rows_per_w, _LANES), jnp.float32),
            pltpu.VMEM((rows_per_w, _LANES), jnp.int32),
            pltpu.SemaphoreType.DMA((2,)),
        ],
    )
    def topk16(probs_hbm, outv_hbm, outi_hbm, buf, macc, rv, ri, sem):
        wid = lax.axis_index("s") * info.num_cores + lax.axis_index("c")
        iota = lax.iota(jnp.int32, _LANES)
        shift_idx = jnp.maximum(iota - 1, 0)

        def insert_loop(v, base, CV, CI, Ts):
            m = v > jnp.zeros((_LANES,), jnp.float32) + Ts

            def wcond(st):
                return jnp.any(st[3])

            def wbody(st):
                CV, CI, Ts, m = st
                fv = plsc.all_reduce_ffs(m)
                x = _dyn_gather(v, fv)
                xi = fv + base
                p = plsc.all_reduce_population_count(CV >= x)
                sv = _dyn_gather(CV, shift_idx)
                si = _dyn_gather(CI, shift_idx)
                CVn = jnp.where(iota < p, CV, jnp.where(iota == p, x, sv))
                CIn = jnp.where(iota < p, CI, jnp.where(iota == p, xi, si))
                Tn = jnp.min(CVn)
                mn = m & (iota != fv) & (v > jnp.zeros((_LANES,), jnp.float32) + Tn)
                return CVn, CIn, Tn, mn

            CV, CI, Ts, _ = lax.while_loop(wcond, wbody, (CV, CI, Ts, m))
            return CV, CI, Ts

        def dma(c, slot):
            row = wid * rows_per_w + c // chunks_per_row
            off = (c % chunks_per_row) * chunk
            return pltpu.make_async_copy(
                probs_hbm.at[row, pl.ds(off, chunk)], buf.at[slot], sem.at[slot]
            )

        dma(0, 0).start()

        def chunk_body(c, carry):
            CV, CI, Ts = carry
            slot = c % 2

            @pl.when(c + 1 < totch)
            def _():
                dma(c + 1, 1 - slot).start()

            dma(c, slot).wait()

            rs = c % chunks_per_row == 0
            CV = jnp.where(rs, jnp.full((_LANES,), -1.0, jnp.float32), CV)
            CI = jnp.where(rs, jnp.zeros((_LANES,), jnp.int32), CI)
            Ts = jnp.where(rs, jnp.float32(-1.0), Ts)
            rowbase = (c % chunks_per_row) * chunk

            def block_body(b, carry2):
                bb = b * blk
                accs = []
                for k in range(sub):
                    acc = buf[slot, pl.ds(bb + k * sub * _LANES, _LANES)]
                    for u in range(1, sub):
                        acc = jnp.maximum(
                            acc, buf[slot, pl.ds(bb + (k * sub + u) * _LANES, _LANES)]
                        )
                    macc[k] = acc
                    accs.append(acc)
                m = accs[0]
                for k in range(1, sub):
                    m = jnp.maximum(m, accs[k])
                s = jnp.max(m)

                def rescan(carry3):
                    def sub_body(k, carry4):
                        CV, CI, Ts = carry4
                        sk = jnp.max(macc[k])

                        def sub_rescan(carry5):
                            def vreg_body(u, carry6):
                                CV, CI, Ts = carry6
                                o = bb + (k * sub + u) * _LANES
                                v = buf[slot, pl.ds(o, _LANES)]
                                mm = v > jnp.zeros((_LANES,), jnp.float32) + Ts
                                return lax.cond(
                                    jnp.any(mm),
                                    lambda a: insert_loop(v, rowbase + o, *a),
                                    lambda a: a,
                                    (CV, CI, Ts),
                                )

                            return lax.fori_loop(0, sub, vreg_body, carry5)

                        return lax.cond(sk > Ts, sub_rescan, lambda a: a, carry4)

                    return lax.fori_loop(0, sub, sub_body, carry3)

                return lax.cond(s > carry2[2], rescan, lambda a: a, carry2)

            CV, CI, Ts = lax.fori_loop(0, blocks_per_chunk, block_body, (CV, CI, Ts))

            @pl.when(c % chunks_per_row == chunks_per_row - 1)
            def _(CV=CV, CI=CI):
                r = c // chunks_per_row
                rv[r] = CV
                ri[r] = CI

            return CV, CI, Ts

        lax.fori_loop(
            0,
            totch,
            chunk_body,
            (
                jnp.full((_LANES,), -1.0, jnp.float32),
                jnp.zeros((_LANES,), jnp.int32),
                jnp.float32(-1.0),
            ),
        )

        out_off = pl.multiple_of(wid * rows_per_w, rows_per_w)
        pltpu.sync_copy(rv, outv_hbm.at[pl.ds(out_off, rows_per_w)])
        pltpu.sync_copy(ri, outi_hbm.at[pl.ds(out_off, rows_per_w)])

    return topk16


def _span_f1(ps, pe, gs, ge):
    ps_f = ps.astype(jnp.float32)
    pe_f = pe.astype(jnp.float32)
    gs_f = gs.astype(jnp.float32)
    ge_f = ge.astype(jnp.float32)
    inter = jnp.maximum(0.0, jnp.minimum(pe_f, ge_f) - jnp.maximum(ps_f, gs_f) + 1.0)
    pred_len = jnp.maximum(pe_f - ps_f + 1.0, 1.0)
    gold_len = jnp.maximum(ge_f - gs_f + 1.0, 1.0)
    prec = inter / pred_len
    rec = inter / gold_len
    f1 = 2.0 * prec * rec / jnp.maximum(prec + rec, 1e-8)
    return jnp.where(ps <= pe, f1, 0.0).astype(jnp.float32)


def kernel(probs, start, end, context_len):
    B, N = probs.shape
    topk16 = _make_topk16(B, N)
    cv, ci = topk16(probs)

    K = 10
    kbest_probs = cv[:, :K]
    kbest = ci[:, :K]
    greedy_idx = ci[:, 0]
    greedy_reward = _span_f1(
        greedy_idx // context_len, greedy_idx % context_len, start, end
    )
    skey = jax.random.key(42)
    indice = jax.random.categorical(skey, jnp.log(kbest_probs + 1e-20), axis=1)
    sample_idx = jnp.take_along_axis(kbest, indice[:, None], axis=1).squeeze(1)
    sample_reward = _span_f1(
        sample_idx // context_len, sample_idx % context_len, start, end
    )
    greedy_better = jnp.clip(greedy_reward - sample_reward, 0.0, 1.0e7)
    sample_better = jnp.clip(sample_reward, 0.0, 1.0e7)
    greedy_loss = -cv[:, 0]
    sample_loss = -jnp.take_along_axis(kbest_probs, indice[:, None], axis=1).squeeze(1)
    total_loss = greedy_better * greedy_loss + sample_better * sample_loss
    return jnp.mean(total_loss)


# speedup vs baseline: 9.8395x; 1.0381x over previous
import functools

import jax
import jax.numpy as jnp
from jax import lax
from jax.experimental import pallas as pl
from jax.experimental.pallas import tpu as pltpu
from jax.experimental.pallas import tpu_sc as plsc

_LANES = 16


def _dyn_gather(src, idx):
    return lax.gather(
        src,
        idx[:, None],
        lax.GatherDimensionNumbers(
            offset_dims=(), collapsed_slice_dims=(0,), start_index_map=(0,)
        ),
        slice_sizes=(1,),
        mode=lax.GatherScatterMode.PROMISE_IN_BOUNDS,
    )


def _tree_max(vs):
    while len(vs) > 1:
        nxt = [jnp.maximum(vs[i], vs[i + 1]) for i in range(0, len(vs) - 1, 2)]
        if len(vs) % 2:
            nxt.append(vs[-1])
        vs = nxt
    return vs[0]


def _make_topk16(B, N, chunk=32768, sub=16, nsub=16):
    info = plsc.get_sparse_core_info()
    nw = info.num_cores * info.num_subcores
    rows_per_w = B // nw
    chunks_per_row = N // chunk
    totch = rows_per_w * chunks_per_row
    sblk = sub * nsub * _LANES
    sblocks_per_chunk = chunk // sblk

    mesh = plsc.VectorSubcoreMesh(core_axis_name="c", subcore_axis_name="s")

    @functools.partial(
        pl.kernel,
        mesh=mesh,
        out_type=[
            jax.ShapeDtypeStruct((B, _LANES), jnp.float32),
            jax.ShapeDtypeStruct((B, _LANES), jnp.int32),
        ],
        compiler_params=pltpu.CompilerParams(needs_layout_passes=False),
        scratch_types=[
            pltpu.VMEM((2, chunk), jnp.float32),
            pltpu.VMEM((rows_per_w, _LANES), jnp.float32),
            pltpu.VMEM((rows_per_w, _LANES), jnp.int32),
            pltpu.SemaphoreType.DMA((2,)),
        ],
    )
    def topk16(probs_hbm, outv_hbm, outi_hbm, buf, rv, ri, sem):
        wid = lax.axis_index("s") * info.num_cores + lax.axis_index("c")
        iota = lax.iota(jnp.int32, _LANES)
        shift_idx = jnp.maximum(iota - 1, 0)
        lane15 = jnp.full((_LANES,), _LANES - 1, jnp.int32)
        zero_f = jnp.zeros((_LANES,), jnp.float32)

        def insert_candidates(v, idxvec, CV, CI, Tv):
            I16 = _dyn_gather(CI, lane15)
            m = (v > Tv) | ((v == Tv) & (idxvec < I16))

            def wcond(st):
                return jnp.any(st[3])

            def wbody(st):
                CV, CI, Tv, m = st
                fv = plsc.all_reduce_ffs(m)
                x = _dyn_gather(v, fv)
                xi = _dyn_gather(idxvec, fv)
                p = plsc.all_reduce_population_count(
                    (CV > x) | ((CV == x) & (CI < xi))
                )
                sv = _dyn_gather(CV, shift_idx)
                si = _dyn_gather(CI, shift_idx)
                CVn = jnp.where(iota < p, CV, jnp.where(iota == p, x, sv))
                CIn = jnp.where(iota < p, CI, jnp.where(iota == p, xi, si))
                Tn = _dyn_gather(CVn, lane15)
                I16n = _dyn_gather(CIn, lane15)
                mn = (
                    m
                    & (iota != fv)
                    & ((v > Tn) | ((v == Tn) & (idxvec < I16n)))
                )
                return CVn, CIn, Tn, mn

            CV, CI, Tv, _ = lax.while_loop(wcond, wbody, (CV, CI, Tv, m))
            return CV, CI, Tv

        def dma(c, slot):
            row = wid * rows_per_w + c // chunks_per_row
            off = (c % chunks_per_row) * chunk
            return pltpu.make_async_copy(
                probs_hbm.at[row, pl.ds(off, chunk)], buf.at[slot], sem.at[slot]
            )

        dma(0, 0).start()

        def chunk_body(c, carry):
            CV, CI, Tv = carry
            slot = c % 2

            @pl.when(c + 1 < totch)
            def _():
                dma(c + 1, 1 - slot).start()

            dma(c, slot).wait()

            rs = c % chunks_per_row == 0
            CV = jnp.where(rs, jnp.full((_LANES,), -1.0, jnp.float32), CV)
            CI = jnp.where(rs, jnp.zeros((_LANES,), jnp.int32), CI)
            Tv = jnp.where(rs, jnp.full((_LANES,), -1.0, jnp.float32), Tv)
            rowbase = (c % chunks_per_row) * chunk
            slotv = jnp.zeros((_LANES,), jnp.int32) + slot

            def sblock_body(sb, carry2):
                CV, CI, Tv = carry2
                sbb = sb * sblk
                S = None
                for k in range(nsub):
                    base = sbb + k * (sub * _LANES)
                    vs = [
                        buf[slot, pl.ds(base + u * _LANES, _LANES)]
                        for u in range(sub)
                    ]
                    acc = _tree_max(vs)
                    cum = plsc.cummax(acc)
                    bk = _dyn_gather(cum, lane15)
                    S = bk if k == 0 else jnp.where(iota == k, bk, S)

                hit = jnp.max(jnp.where(S >= Tv, 1, 0))

                def rescan(carry3):
                    CV, CI, Tv = carry3
                    m = S >= Tv

                    def sub_cond(st):
                        return jnp.any(st[3])

                    def sub_body(st):
                        CV, CI, Tv, m = st
                        lv = plsc.all_reduce_ffs(m)
                        ls = jnp.max(lv)
                        base = sbb + ls * (sub * _LANES)
                        vs = [
                            buf[slot, pl.ds(base + u * _LANES, _LANES)]
                            for u in range(sub)
                        ]
                        acc = _tree_max(vs)
                        m2 = acc >= Tv

                        def lane_cond(st2):
                            return jnp.any(st2[3])

                        def lane_body(st2):
                            CV, CI, Tv, m2 = st2
                            qv = plsc.all_reduce_ffs(m2)
                            colidx = base + qv + iota * _LANES
                            col = plsc.load_gather(buf, [slotv, colidx])
                            CV, CI, Tv = insert_candidates(
                                col, rowbase + colidx, CV, CI, Tv
                            )
                            m2 = m2 & (iota != qv) & (acc >= Tv)
                            return CV, CI, Tv, m2

                        CV, CI, Tv, _ = lax.while_loop(
                            lane_cond, lane_body, (CV, CI, Tv, m2)
                        )
                        m = m & (iota != lv) & (S >= Tv)
                        return CV, CI, Tv, m

                    CV, CI, Tv, _ = lax.while_loop(
                        sub_cond, sub_body, (CV, CI, Tv, m)
                    )
                    return CV, CI, Tv

                return lax.cond(hit > 0, rescan, lambda a: a, (CV, CI, Tv))

            CV, CI, Tv = lax.fori_loop(0, sblocks_per_chunk, sblock_body, (CV, CI, Tv))

            @pl.when(c % chunks_per_row == chunks_per_row - 1)
            def _(CV=CV, CI=CI):
                r = c // chunks_per_row
                rv[r] = CV
                ri[r] = CI

            return CV, CI, Tv

        lax.fori_loop(
            0,
            totch,
            chunk_body,
            (
                jnp.full((_LANES,), -1.0, jnp.float32),
                jnp.zeros((_LANES,), jnp.int32),
                jnp.full((_LANES,), -1.0, jnp.float32),
            ),
        )

        out_off = pl.multiple_of(wid * rows_per_w, rows_per_w)
        pltpu.sync_copy(rv, outv_hbm.at[pl.ds(out_off, rows_per_w)])
        pltpu.sync_copy(ri, outi_hbm.at[pl.ds(out_off, rows_per_w)])

    return topk16


def _span_f1(ps, pe, gs, ge):
    ps_f = ps.astype(jnp.float32)
    pe_f = pe.astype(jnp.float32)
    gs_f = gs.astype(jnp.float32)
    ge_f = ge.astype(jnp.float32)
    inter = jnp.maximum(0.0, jnp.minimum(pe_f, ge_f) - jnp.maximum(ps_f, gs_f) + 1.0)
    pred_len = jnp.maximum(pe_f - ps_f + 1.0, 1.0)
    gold_len = jnp.maximum(ge_f - gs_f + 1.0, 1.0)
    prec = inter / pred_len
    rec = inter / gold_len
    f1 = 2.0 * prec * rec / jnp.maximum(prec + rec, 1e-8)
    return jnp.where(ps <= pe, f1, 0.0).astype(jnp.float32)


def kernel(probs, start, end, context_len):
    B, N = probs.shape
    topk16 = _make_topk16(B, N)
    cv, ci = topk16(probs)

    K = 10
    kbest_probs = cv[:, :K]
    kbest = ci[:, :K]
    greedy_idx = ci[:, 0]
    greedy_reward = _span_f1(
        greedy_idx // context_len, greedy_idx % context_len, start, end
    )
    skey = jax.random.key(42)
    indice = jax.random.categorical(skey, jnp.log(kbest_probs + 1e-20), axis=1)
    sample_idx = jnp.take_along_axis(kbest, indice[:, None], axis=1).squeeze(1)
    sample_reward = _span_f1(
        sample_idx // context_len, sample_idx % context_len, start, end
    )
    greedy_better = jnp.clip(greedy_reward - sample_reward, 0.0, 1.0e7)
    sample_better = jnp.clip(sample_reward, 0.0, 1.0e7)
    greedy_loss = -cv[:, 0]
    sample_loss = -jnp.take_along_axis(kbest_probs, indice[:, None], axis=1).squeeze(1)
    total_loss = greedy_better * greedy_loss + sample_better * sample_loss
    return jnp.mean(total_loss)


# speedup vs baseline: 19.6701x; 1.9991x over previous
import functools

import jax
import jax.numpy as jnp
from jax import lax
from jax.experimental import pallas as pl
from jax.experimental.pallas import tpu as pltpu
from jax.experimental.pallas import tpu_sc as plsc

_LANES = 16


def _scal(x):
    return lax.squeeze(lax.slice(x, (0,), (1,)), dimensions=(0,))


def _dyn_gather(src, idx):
    return lax.gather(
        src,
        idx[:, None],
        lax.GatherDimensionNumbers(
            offset_dims=(), collapsed_slice_dims=(0,), start_index_map=(0,)
        ),
        slice_sizes=(1,),
        mode=lax.GatherScatterMode.PROMISE_IN_BOUNDS,
    )


def _tree_max(vs):
    while len(vs) > 1:
        nxt = [jnp.maximum(vs[i], vs[i + 1]) for i in range(0, len(vs) - 1, 2)]
        if len(vs) % 2:
            nxt.append(vs[-1])
        vs = nxt
    return vs[0]


def _make_topk16(B, N, chunk=32768, sub=16, nsub=16):
    info = plsc.get_sparse_core_info()
    nw = info.num_cores * info.num_subcores
    rows_per_w = B // nw
    chunks_per_row = N // chunk
    totch = rows_per_w * chunks_per_row
    sblk = sub * nsub * _LANES
    sblocks_per_chunk = chunk // sblk

    mesh = plsc.VectorSubcoreMesh(core_axis_name="c", subcore_axis_name="s")

    @functools.partial(
        pl.kernel,
        mesh=mesh,
        out_type=[
            jax.ShapeDtypeStruct((B, _LANES), jnp.float32),
            jax.ShapeDtypeStruct((B, _LANES), jnp.int32),
        ],
        compiler_params=pltpu.CompilerParams(needs_layout_passes=False),
        scratch_types=[
            pltpu.VMEM((2, chunk), jnp.float32),
            pltpu.VMEM((chunk // (sub * nsub * _LANES), _LANES), jnp.float32),
            pltpu.VMEM((rows_per_w, _LANES), jnp.float32),
            pltpu.VMEM((rows_per_w, _LANES), jnp.int32),
            pltpu.SemaphoreType.DMA((2,)),
        ],
    )
    def topk16(probs_hbm, outv_hbm, outi_hbm, buf, sumref, rv, ri, sem):
        wid = lax.axis_index("s") * info.num_cores + lax.axis_index("c")
        iota = lax.iota(jnp.int32, _LANES)
        shift_idx = jnp.maximum(iota - 1, 0)
        lane15 = jnp.full((_LANES,), _LANES - 1, jnp.int32)
        zero_f = jnp.zeros((_LANES,), jnp.float32)

        def insert_candidates(v, idxvec, CV, CI, Tv):
            I16 = _dyn_gather(CI, lane15)
            m = (v > Tv) | ((v == Tv) & (idxvec < I16))

            def wcond(st):
                return _scal(plsc.all_reduce_population_count(st[3])) > 0

            def wbody(st):
                CV, CI, Tv, m = st
                fv = plsc.all_reduce_ffs(m)
                x = _dyn_gather(v, fv)
                xi = _dyn_gather(idxvec, fv)
                p = plsc.all_reduce_population_count(
                    (CV > x) | ((CV == x) & (CI < xi))
                )
                sv = _dyn_gather(CV, shift_idx)
                si = _dyn_gather(CI, shift_idx)
                CVn = jnp.where(iota < p, CV, jnp.where(iota == p, x, sv))
                CIn = jnp.where(iota < p, CI, jnp.where(iota == p, xi, si))
                Tn = _dyn_gather(CVn, lane15)
                I16n = _dyn_gather(CIn, lane15)
                mn = (
                    m
                    & (iota != fv)
                    & ((v > Tn) | ((v == Tn) & (idxvec < I16n)))
                )
                return CVn, CIn, Tn, mn

            CV, CI, Tv, _ = lax.while_loop(wcond, wbody, (CV, CI, Tv, m))
            return CV, CI, Tv

        def dma(c, slot):
            row = wid * rows_per_w + c // chunks_per_row
            off = (c % chunks_per_row) * chunk
            return pltpu.make_async_copy(
                probs_hbm.at[row, pl.ds(off, chunk)], buf.at[slot], sem.at[slot]
            )

        dma(0, 0).start()

        def chunk_body(c, carry):
            CV, CI, Tv = carry
            slot = c % 2

            dma(c, slot).wait()

            rs = c % chunks_per_row == 0
            CV = jnp.where(rs, jnp.full((_LANES,), -1.0, jnp.float32), CV)
            CI = jnp.where(rs, jnp.zeros((_LANES,), jnp.int32), CI)
            Tv = jnp.where(rs, jnp.full((_LANES,), -1.0, jnp.float32), Tv)
            rowbase = (c % chunks_per_row) * chunk
            slotv = jnp.zeros((_LANES,), jnp.int32) + slot

            cn = jnp.minimum(c + 1, totch - 1)
            nrow = wid * rows_per_w + cn // chunks_per_row
            noff = (cn % chunks_per_row) * chunk
            piece = chunk // (sblocks_per_chunk * (nsub // 4))

            def group_body(g, S):
                for j in range(4):
                    base = g * (4 * sub * _LANES) + j * (sub * _LANES)
                    vs = [
                        buf[slot, pl.ds(base + u * _LANES, _LANES)]
                        for u in range(sub)
                    ]
                    acc = _tree_max(vs)
                    cum = plsc.cummax(acc)
                    bk = _dyn_gather(cum, lane15)
                    S = jnp.where(iota == (g * 4 + j) % nsub, bk, S)
                pltpu.make_async_copy(
                    probs_hbm.at[nrow, pl.ds(noff + g * piece, piece)],
                    buf.at[1 - slot, pl.ds(g * piece, piece)],
                    sem.at[1 - slot],
                ).start()
                sumref[g // 4] = S
                return S

            lax.fori_loop(
                0, sblocks_per_chunk * (nsub // 4), group_body, zero_f
            )

            def sblock_body(sb, carry2):
                CV, CI, Tv = carry2
                sbb = sb * sblk
                S = sumref[sb]
                hit = _scal(plsc.all_reduce_population_count(S >= Tv))

                def rescan(carry3):
                    CV, CI, Tv = carry3
                    m = S >= Tv

                    def sub_cond(st):
                        return _scal(plsc.all_reduce_population_count(st[3])) > 0

                    def sub_body(st):
                        CV, CI, Tv, m = st
                        lv = plsc.all_reduce_ffs(m)
                        ls = _scal(lv)
                        base = sbb + ls * (sub * _LANES)
                        vs = [
                            buf[slot, pl.ds(base + u * _LANES, _LANES)]
                            for u in range(sub)
                        ]
                        acc = _tree_max(vs)
                        m2 = acc >= Tv

                        def lane_cond(st2):
                            return _scal(plsc.all_reduce_population_count(st2[3])) > 0

                        def lane_body(st2):
                            CV, CI, Tv, m2 = st2
                            qv = plsc.all_reduce_ffs(m2)
                            colidx = base + qv + iota * _LANES
                            col = plsc.load_gather(buf, [slotv, colidx])
                            CV, CI, Tv = insert_candidates(
                                col, rowbase + colidx, CV, CI, Tv
                            )
                            m2 = m2 & (iota != qv) & (acc >= Tv)
                            return CV, CI, Tv, m2

                        CV, CI, Tv, _ = lax.while_loop(
                            lane_cond, lane_body, (CV, CI, Tv, m2)
                        )
                        m = m & (iota != lv) & (S >= Tv)
                        return CV, CI, Tv, m

                    CV, CI, Tv, _ = lax.while_loop(
                        sub_cond, sub_body, (CV, CI, Tv, m)
                    )
                    return CV, CI, Tv

                return lax.cond(hit > 0, rescan, lambda a: a, (CV, CI, Tv))

            CV, CI, Tv = lax.fori_loop(0, sblocks_per_chunk, sblock_body, (CV, CI, Tv))

            @pl.when(c % chunks_per_row == chunks_per_row - 1)
            def _(CV=CV, CI=CI):
                r = c // chunks_per_row
                rv[r] = CV
                ri[r] = CI

            return CV, CI, Tv

        lax.fori_loop(
            0,
            totch,
            chunk_body,
            (
                jnp.full((_LANES,), -1.0, jnp.float32),
                jnp.zeros((_LANES,), jnp.int32),
                jnp.full((_LANES,), -1.0, jnp.float32),
            ),
        )

        dma(totch - 1, (totch % 2)).wait()

        out_off = pl.multiple_of(wid * rows_per_w, rows_per_w)
        pltpu.sync_copy(rv, outv_hbm.at[pl.ds(out_off, rows_per_w)])
        pltpu.sync_copy(ri, outi_hbm.at[pl.ds(out_off, rows_per_w)])

    return topk16


def _span_f1(ps, pe, gs, ge):
    ps_f = ps.astype(jnp.float32)
    pe_f = pe.astype(jnp.float32)
    gs_f = gs.astype(jnp.float32)
    ge_f = ge.astype(jnp.float32)
    inter = jnp.maximum(0.0, jnp.minimum(pe_f, ge_f) - jnp.maximum(ps_f, gs_f) + 1.0)
    pred_len = jnp.maximum(pe_f - ps_f + 1.0, 1.0)
    gold_len = jnp.maximum(ge_f - gs_f + 1.0, 1.0)
    prec = inter / pred_len
    rec = inter / gold_len
    f1 = 2.0 * prec * rec / jnp.maximum(prec + rec, 1e-8)
    return jnp.where(ps <= pe, f1, 0.0).astype(jnp.float32)


def kernel(probs, start, end, context_len):
    B, N = probs.shape
    topk16 = _make_topk16(B, N)
    cv, ci = topk16(probs)

    K = 10
    kbest_probs = cv[:, :K]
    kbest = ci[:, :K]
    greedy_idx = ci[:, 0]
    greedy_reward = _span_f1(
        greedy_idx // context_len, greedy_idx % context_len, start, end
    )
    skey = jax.random.key(42)
    indice = jax.random.categorical(skey, jnp.log(kbest_probs + 1e-20), axis=1)
    sample_idx = jnp.take_along_axis(kbest, indice[:, None], axis=1).squeeze(1)
    sample_reward = _span_f1(
        sample_idx // context_len, sample_idx % context_len, start, end
    )
    greedy_better = jnp.clip(greedy_reward - sample_reward, 0.0, 1.0e7)
    sample_better = jnp.clip(sample_reward, 0.0, 1.0e7)
    greedy_loss = -cv[:, 0]
    sample_loss = -jnp.take_along_axis(kbest_probs, indice[:, None], axis=1).squeeze(1)
    total_loss = greedy_better * greedy_loss + sample_better * sample_loss
    return jnp.mean(total_loss)
